# baseline (speedup 1.0000x reference)
import functools

import jax
import jax.numpy as jnp
from jax import lax
from jax.experimental import pallas as pl
from jax.experimental.pallas import tpu as pltpu
from jax.experimental.pallas import tpu_sc as plsc

BATCH = 4096
MAXLEN = 200
EMBED = 32

_NC = 2
_NS = 16
_NW = _NC * _NS
_TT = MAXLEN // 8
_TCH = 4
_NCH = MAXLEN // _TCH
_NBLK = _TCH * (EMBED // 8)


def _issue_gather(tok_hbm, idx_all, rows_b, sem, t0):
    for tl in range(_TCH):
        pltpu.async_copy(
            tok_hbm.at[idx_all.at[t0 + tl]],
            rows_b.at[pl.ds(tl * 128, 128)],
            sem,
        )


def _wait_gather(tok_hbm, rows_b, sem):
    pltpu.make_async_copy(tok_hbm.at[pl.ds(0, _TCH * 128)], rows_b, sem).wait()


def _transpose_add(rows_b, pos_v, outb, t0):
    lanes = lax.iota(jnp.int32, 16)
    in_base = (lanes % 8) * 128
    for tl in range(_TCH):
        t = t0 + tl
        p0 = pos_v[t, pl.ds(0, 16)]
        p1 = pos_v[t, pl.ds(16, 16)]
        blk0 = tl * 4 + lanes // 8
        blk1 = blk0 + 2

        def body(b, idx_in):
            r0 = rows_b[tl * 128 + b, pl.ds(0, 16)]
            r1 = rows_b[tl * 128 + b, pl.ds(16, 16)]
            plsc.store_scatter(outb, [blk0, idx_in], r0 + p0)
            plsc.store_scatter(outb, [blk1, idx_in], r1 + p1)
            return idx_in + 1

        lax.fori_loop(0, 128, body, in_base, unroll=4)


def _emb_body(x_hbm, tok_hbm, pos_hbm, out_hbm,
              idx_all, pos_v, rows2, out2, gsem0, gsem1, osem0, osem1):
    w = lax.axis_index("s") * _NC + lax.axis_index("c")

    rows_b0 = rows2.at[0]
    rows_b1 = rows2.at[1]
    outb0 = out2.at[0]
    outb1 = out2.at[1]

    pltpu.sync_copy(pos_hbm, pos_v)
    for tt in range(_TT):
        pltpu.sync_copy(x_hbm.at[tt, w], idx_all.at[pl.ds(tt * 8, 8)])

    _issue_gather(tok_hbm, idx_all, rows_b0, gsem0, 0)

    def outer(j, carry):
        ta = 2 * j * _TCH
        tb = ta + _TCH

        _wait_gather(tok_hbm, rows_b0, gsem0)

        @pl.when(j > 0)
        def _():
            pltpu.make_async_copy(
                outb1, out_hbm.at[pl.ds(0, _NBLK), w], osem1
            ).wait()

        _issue_gather(tok_hbm, idx_all, rows_b1, gsem1, tb)
        _transpose_add(rows_b0, pos_v, outb0, ta)
        pltpu.async_copy(outb0, out_hbm.at[pl.ds(ta * 4, _NBLK), w], osem0)

        _wait_gather(tok_hbm, rows_b1, gsem1)
        pltpu.make_async_copy(outb0, out_hbm.at[pl.ds(0, _NBLK), w], osem0).wait()

        @pl.when(j < _NCH // 2 - 1)
        def _():
            _issue_gather(tok_hbm, idx_all, rows_b0, gsem0, tb + _TCH)

        _transpose_add(rows_b1, pos_v, outb1, tb)
        pltpu.async_copy(outb1, out_hbm.at[pl.ds(tb * 4, _NBLK), w], osem1)
        return carry

    lax.fori_loop(0, _NCH // 2, outer, 0)

    pltpu.make_async_copy(outb1, out_hbm.at[pl.ds(0, _NBLK), w], osem1).wait()


@jax.jit
def _emb_call(x4, token_emb, pos_emb):
    mesh = plsc.VectorSubcoreMesh(core_axis_name="c", subcore_axis_name="s")
    k = functools.partial(
        pl.kernel,
        mesh=mesh,
        out_type=jax.ShapeDtypeStruct((MAXLEN * 4, _NW, 1024), jnp.float32),
        scratch_types=[
            pltpu.VMEM((MAXLEN, 128), jnp.int32),
            pltpu.VMEM((MAXLEN, EMBED), jnp.float32),
            pltpu.VMEM((2, _TCH * 128, EMBED), jnp.float32),
            pltpu.VMEM((2, _NBLK, 1024), jnp.float32),
            pltpu.SemaphoreType.DMA,
            pltpu.SemaphoreType.DMA,
            pltpu.SemaphoreType.DMA,
            pltpu.SemaphoreType.DMA,
        ],
        compiler_params=pltpu.CompilerParams(
            use_tc_tiling_on_sc=False, needs_layout_passes=False
        ),
    )(_emb_body)
    return k(x4, token_emb, pos_emb)


def kernel(x, token_emb, pos_emb):
    x4 = (
        x.astype(jnp.int32)
        .reshape(_NW, 128, _TT, 8)
        .transpose(2, 0, 3, 1)
    )
    out = _emb_call(x4, token_emb, pos_emb)
    return (
        out.reshape(MAXLEN, 4, _NW, 8, 128)
        .transpose(2, 4, 0, 1, 3)
        .reshape(BATCH, MAXLEN, EMBED)
    )


# speedup vs baseline: 2.4978x; 2.4978x over previous
import functools

import jax
import jax.numpy as jnp
from jax import lax
from jax.experimental import pallas as pl
from jax.experimental.pallas import tpu as pltpu
from jax.experimental.pallas import tpu_sc as plsc

BATCH = 4096
MAXLEN = 200
EMBED = 32

_NC = 2
_NS = 16
_NW = _NC * _NS
_TT = MAXLEN // 8
_TCH = 4
_NCH = MAXLEN // _TCH
_NBLK = _TCH * (EMBED // 8)
_PSTR = 129


def _issue_gather(tok_hbm, idx_all, rows_b, sem, t0):
    for tl in range(_TCH):
        pltpu.async_copy(
            tok_hbm.at[idx_all.at[t0 + tl]],
            rows_b.at[pl.ds(tl * 128, 128)],
            sem,
        )


def _wait_gather(tok_hbm, rows_b, sem):
    pltpu.make_async_copy(tok_hbm.at[pl.ds(0, _TCH * 128)], rows_b, sem).wait()


def _transpose_add(rows_b, pos_v, outb, t0):
    lanes = lax.iota(jnp.int32, 16)
    ev = lanes % 8
    zeros = lanes * 0
    for tl in range(_TCH):
        t = t0 + tl
        p0 = pos_v[t, pl.ds(0, 16)]
        p1 = pos_v[t, pl.ds(16, 16)]
        blk0 = tl * 4 + lanes // 8
        blk1 = blk0 + 2

        def body(b, idx_in):
            r0 = rows_b[tl * 128 + b, pl.ds(0, 16)]
            r1 = rows_b[tl * 128 + b, pl.ds(16, 16)]
            plsc.store_scatter(outb, [blk0, ev, idx_in], r0 + p0)
            plsc.store_scatter(outb, [blk1, ev, idx_in], r1 + p1)
            return idx_in + 1

        lax.fori_loop(0, 128, body, zeros, unroll=4)


def _emb_body(x_hbm, tok_hbm, pos_hbm, out_hbm,
              idx_all, pos_v, rows2, out2, gsem0, gsem1, osem0, osem1):
    w = lax.axis_index("s") * _NC + lax.axis_index("c")

    rows_b0 = rows2.at[0]
    rows_b1 = rows2.at[1]
    outb0 = out2.at[0]
    outb1 = out2.at[1]

    pltpu.sync_copy(pos_hbm, pos_v)
    for tt in range(_TT):
        pltpu.sync_copy(x_hbm.at[tt, w], idx_all.at[pl.ds(tt * 8, 8)])

    _issue_gather(tok_hbm, idx_all, rows_b0, gsem0, 0)

    def outer(j, carry):
        ta = 2 * j * _TCH
        tb = ta + _TCH

        _wait_gather(tok_hbm, rows_b0, gsem0)

        @pl.when(j > 0)
        def _():
            pltpu.make_async_copy(
                outb1.at[:, :, pl.ds(0, 128)],
                out_hbm.at[pl.ds(0, _NBLK), w],
                osem1,
            ).wait()

        _issue_gather(tok_hbm, idx_all, rows_b1, gsem1, tb)
        _transpose_add(rows_b0, pos_v, outb0, ta)
        pltpu.async_copy(
            outb0.at[:, :, pl.ds(0, 128)],
            out_hbm.at[pl.ds(ta * 4, _NBLK), w],
            osem0,
        )

        _wait_gather(tok_hbm, rows_b1, gsem1)
        pltpu.make_async_copy(
            outb0.at[:, :, pl.ds(0, 128)],
            out_hbm.at[pl.ds(0, _NBLK), w],
            osem0,
        ).wait()

        @pl.when(j < _NCH // 2 - 1)
        def _():
            _issue_gather(tok_hbm, idx_all, rows_b0, gsem0, tb + _TCH)

        _transpose_add(rows_b1, pos_v, outb1, tb)
        pltpu.async_copy(
            outb1.at[:, :, pl.ds(0, 128)],
            out_hbm.at[pl.ds(tb * 4, _NBLK), w],
            osem1,
        )
        return carry

    lax.fori_loop(0, _NCH // 2, outer, 0)

    pltpu.make_async_copy(
        outb1.at[:, :, pl.ds(0, 128)],
        out_hbm.at[pl.ds(0, _NBLK), w],
        osem1,
    ).wait()


@jax.jit
def _emb_call(x4, token_emb, pos_emb):
    mesh = plsc.VectorSubcoreMesh(core_axis_name="c", subcore_axis_name="s")
    k = functools.partial(
        pl.kernel,
        mesh=mesh,
        out_type=jax.ShapeDtypeStruct((MAXLEN * 4, _NW, 8, 128), jnp.float32),
        scratch_types=[
            pltpu.VMEM((MAXLEN, 128), jnp.int32),
            pltpu.VMEM((MAXLEN, EMBED), jnp.float32),
            pltpu.VMEM((2, _TCH * 128, EMBED), jnp.float32),
            pltpu.VMEM((2, _NBLK, 8, _PSTR), jnp.float32),
            pltpu.SemaphoreType.DMA,
            pltpu.SemaphoreType.DMA,
            pltpu.SemaphoreType.DMA,
            pltpu.SemaphoreType.DMA,
        ],
        compiler_params=pltpu.CompilerParams(
            use_tc_tiling_on_sc=False, needs_layout_passes=False
        ),
    )(_emb_body)
    return k(x4, token_emb, pos_emb)


def kernel(x, token_emb, pos_emb):
    x4 = (
        x.astype(jnp.int32)
        .reshape(_NW, 128, _TT, 8)
        .transpose(2, 0, 3, 1)
    )
    out = _emb_call(x4, token_emb, pos_emb)
    return (
        out.reshape(MAXLEN, 4, _NW, 8, 128)
        .transpose(2, 4, 0, 1, 3)
        .reshape(BATCH, MAXLEN, EMBED)
    )


# speedup vs baseline: 3.7182x; 1.4886x over previous
import functools

import jax
import jax.numpy as jnp
from jax import lax
from jax.experimental import pallas as pl
from jax.experimental.pallas import tpu as pltpu
from jax.experimental.pallas import tpu_sc as plsc

BATCH = 4096
MAXLEN = 200
EMBED = 32

_NC = 2
_NS = 16
_NW = _NC * _NS
_TT = MAXLEN // 8
_TCH = 4
_NCH = MAXLEN // _TCH
_NBLK = _TCH * (EMBED // 8)
_PSTR = 129


def _issue_gather(tok_hbm, idx_all, rows_b, sem, t0):
    for tl in range(_TCH):
        pltpu.async_copy(
            tok_hbm.at[idx_all.at[t0 + tl]],
            rows_b.at[pl.ds(tl * 128, 128)],
            sem,
        )


def _wait_gather(tok_hbm, rows_b, sem):
    pltpu.make_async_copy(tok_hbm.at[pl.ds(0, _TCH * 128)], rows_b, sem).wait()


def _transpose_add(rows_b, pos_v, outb, t0):
    lanes = lax.iota(jnp.int32, 16)
    ev = lanes % 8
    zeros = lanes * 0
    for tl in range(_TCH):
        t = t0 + tl
        p0 = pos_v[t, pl.ds(0, 16)]
        p1 = pos_v[t, pl.ds(16, 16)]
        blk0 = tl * 4 + lanes // 8
        blk1 = blk0 + 2

        @plsc.parallel_loop(0, 128, unroll=8)
        def _(b):
            r0 = rows_b[tl * 128 + b, pl.ds(0, 16)]
            r1 = rows_b[tl * 128 + b, pl.ds(16, 16)]
            bv = zeros + b
            plsc.store_scatter(outb, [blk0, ev, bv], r0 + p0)
            plsc.store_scatter(outb, [blk1, ev, bv], r1 + p1)


def _emb_body(x_hbm, tok_hbm, pos_hbm, out_hbm,
              idx_all, pos_v, rows2, out2, gsem0, gsem1, osem0, osem1):
    w = lax.axis_index("s") * _NC + lax.axis_index("c")

    rows_b0 = rows2.at[0]
    rows_b1 = rows2.at[1]
    outb0 = out2.at[0]
    outb1 = out2.at[1]

    pltpu.sync_copy(pos_hbm, pos_v)
    for tt in range(_TT):
        pltpu.sync_copy(x_hbm.at[tt, w], idx_all.at[pl.ds(tt * 8, 8)])

    _issue_gather(tok_hbm, idx_all, rows_b0, gsem0, 0)

    def outer(j, carry):
        ta = 2 * j * _TCH
        tb = ta + _TCH

        _wait_gather(tok_hbm, rows_b0, gsem0)

        @pl.when(j > 0)
        def _():
            pltpu.make_async_copy(
                outb1.at[:, :, pl.ds(0, 128)],
                out_hbm.at[pl.ds(0, _NBLK), w],
                osem1,
            ).wait()

        _issue_gather(tok_hbm, idx_all, rows_b1, gsem1, tb)
        _transpose_add(rows_b0, pos_v, outb0, ta)
        pltpu.async_copy(
            outb0.at[:, :, pl.ds(0, 128)],
            out_hbm.at[pl.ds(ta * 4, _NBLK), w],
            osem0,
        )

        _wait_gather(tok_hbm, rows_b1, gsem1)
        pltpu.make_async_copy(
            outb0.at[:, :, pl.ds(0, 128)],
            out_hbm.at[pl.ds(0, _NBLK), w],
            osem0,
        ).wait()

        @pl.when(j < _NCH // 2 - 1)
        def _():
            _issue_gather(tok_hbm, idx_all, rows_b0, gsem0, tb + _TCH)

        _transpose_add(rows_b1, pos_v, outb1, tb)
        pltpu.async_copy(
            outb1.at[:, :, pl.ds(0, 128)],
            out_hbm.at[pl.ds(tb * 4, _NBLK), w],
            osem1,
        )
        return carry

    lax.fori_loop(0, _NCH // 2, outer, 0)

    pltpu.make_async_copy(
        outb1.at[:, :, pl.ds(0, 128)],
        out_hbm.at[pl.ds(0, _NBLK), w],
        osem1,
    ).wait()


@jax.jit
def _emb_call(x4, token_emb, pos_emb):
    mesh = plsc.VectorSubcoreMesh(core_axis_name="c", subcore_axis_name="s")
    k = functools.partial(
        pl.kernel,
        mesh=mesh,
        out_type=jax.ShapeDtypeStruct((MAXLEN * 4, _NW, 8, 128), jnp.float32),
        scratch_types=[
            pltpu.VMEM((MAXLEN, 128), jnp.int32),
            pltpu.VMEM((MAXLEN, EMBED), jnp.float32),
            pltpu.VMEM((2, _TCH * 128, EMBED), jnp.float32),
            pltpu.VMEM((2, _NBLK, 8, _PSTR), jnp.float32),
            pltpu.SemaphoreType.DMA,
            pltpu.SemaphoreType.DMA,
            pltpu.SemaphoreType.DMA,
            pltpu.SemaphoreType.DMA,
        ],
        compiler_params=pltpu.CompilerParams(
            use_tc_tiling_on_sc=False, needs_layout_passes=False
        ),
    )(_emb_body)
    return k(x4, token_emb, pos_emb)


def kernel(x, token_emb, pos_emb):
    x4 = (
        x.astype(jnp.int32)
        .reshape(_NW, 128, _TT, 8)
        .transpose(2, 0, 3, 1)
    )
    out = _emb_call(x4, token_emb, pos_emb)
    return (
        out.reshape(MAXLEN, 4, _NW, 8, 128)
        .transpose(2, 4, 0, 1, 3)
        .reshape(BATCH, MAXLEN, EMBED)
    )


# speedup vs baseline: 4.0257x; 1.0827x over previous
import functools

import jax
import jax.numpy as jnp
from jax import lax
from jax.experimental import pallas as pl
from jax.experimental.pallas import tpu as pltpu
from jax.experimental.pallas import tpu_sc as plsc

BATCH = 4096
MAXLEN = 200
EMBED = 32

_NC = 2
_NS = 16
_NW = _NC * _NS
_TT = MAXLEN // 8
_TCH = 4
_NCH = MAXLEN // _TCH
_NBLK = _TCH * (EMBED // 8)
_PSTR = 129


def _issue_gather(tok_hbm, idx_all, rows_b, sem, t0):
    for tl in range(_TCH):
        pltpu.async_copy(
            tok_hbm.at[idx_all.at[t0 + tl]],
            rows_b.at[pl.ds(tl * 128, 128)],
            sem,
        )


def _wait_gather(tok_hbm, rows_b, sem):
    pltpu.make_async_copy(tok_hbm.at[pl.ds(0, _TCH * 128)], rows_b, sem).wait()


def _transpose_add(rows_b, pos_v, outb, t0):
    lanes = lax.iota(jnp.int32, 16)
    ev = lanes % 8
    zeros = lanes * 0
    for tl in range(_TCH):
        t = t0 + tl
        p0 = pos_v[t, pl.ds(0, 16)]
        p1 = pos_v[t, pl.ds(16, 16)]
        blk0 = tl * 4 + lanes // 8
        blk1 = blk0 + 2

        @plsc.parallel_loop(0, 128, unroll=8)
        def _(b):
            r0 = rows_b[tl * 128 + b, pl.ds(0, 16)]
            r1 = rows_b[tl * 128 + b, pl.ds(16, 16)]
            bv = zeros + b
            plsc.store_scatter(outb, [blk0, ev, bv], r0 + p0)
            plsc.store_scatter(outb, [blk1, ev, bv], r1 + p1)


def _emb_body(x_hbm, tok_hbm, pos_hbm, out_hbm,
              idx_all, pos_v, rows2, out2, gsem0, gsem1, osem0, osem1):
    w = lax.axis_index("s") * _NC + lax.axis_index("c")

    rows_b0 = rows2.at[0]
    rows_b1 = rows2.at[1]
    outb0 = out2.at[0]
    outb1 = out2.at[1]

    pltpu.async_copy(pos_hbm, pos_v, osem0)
    for tt in range(_TT):
        pltpu.async_copy(x_hbm.at[tt, w], idx_all.at[pl.ds(tt * 8, 8)], gsem0)
    for tt in range(_TT):
        pltpu.make_async_copy(
            x_hbm.at[tt, w], idx_all.at[pl.ds(tt * 8, 8)], gsem0
        ).wait()
    pltpu.make_async_copy(pos_hbm, pos_v, osem0).wait()

    _issue_gather(tok_hbm, idx_all, rows_b0, gsem0, 0)

    def outer(j, carry):
        ta = 2 * j * _TCH
        tb = ta + _TCH

        _wait_gather(tok_hbm, rows_b0, gsem0)

        @pl.when(j > 0)
        def _():
            pltpu.make_async_copy(
                outb1.at[:, :, pl.ds(0, 128)],
                out_hbm.at[pl.ds(0, _NBLK), w],
                osem1,
            ).wait()

        _issue_gather(tok_hbm, idx_all, rows_b1, gsem1, tb)
        _transpose_add(rows_b0, pos_v, outb0, ta)
        pltpu.async_copy(
            outb0.at[:, :, pl.ds(0, 128)],
            out_hbm.at[pl.ds(ta * 4, _NBLK), w],
            osem0,
        )

        _wait_gather(tok_hbm, rows_b1, gsem1)
        pltpu.make_async_copy(
            outb0.at[:, :, pl.ds(0, 128)],
            out_hbm.at[pl.ds(0, _NBLK), w],
            osem0,
        ).wait()

        @pl.when(j < _NCH // 2 - 1)
        def _():
            _issue_gather(tok_hbm, idx_all, rows_b0, gsem0, tb + _TCH)

        _transpose_add(rows_b1, pos_v, outb1, tb)
        pltpu.async_copy(
            outb1.at[:, :, pl.ds(0, 128)],
            out_hbm.at[pl.ds(tb * 4, _NBLK), w],
            osem1,
        )
        return carry

    lax.fori_loop(0, _NCH // 2, outer, 0)

    pltpu.make_async_copy(
        outb1.at[:, :, pl.ds(0, 128)],
        out_hbm.at[pl.ds(0, _NBLK), w],
        osem1,
    ).wait()


@jax.jit
def _emb_call(x4, token_emb, pos_emb):
    mesh = plsc.VectorSubcoreMesh(core_axis_name="c", subcore_axis_name="s")
    k = functools.partial(
        pl.kernel,
        mesh=mesh,
        out_type=jax.ShapeDtypeStruct((MAXLEN * 4, _NW, 8, 128), jnp.float32),
        scratch_types=[
            pltpu.VMEM((MAXLEN, 128), jnp.int32),
            pltpu.VMEM((MAXLEN, EMBED), jnp.float32),
            pltpu.VMEM((2, _TCH * 128, EMBED), jnp.float32),
            pltpu.VMEM((2, _NBLK, 8, _PSTR), jnp.float32),
            pltpu.SemaphoreType.DMA,
            pltpu.SemaphoreType.DMA,
            pltpu.SemaphoreType.DMA,
            pltpu.SemaphoreType.DMA,
        ],
        compiler_params=pltpu.CompilerParams(
            use_tc_tiling_on_sc=False, needs_layout_passes=False
        ),
    )(_emb_body)
    return k(x4, token_emb, pos_emb)


def kernel(x, token_emb, pos_emb):
    x4 = (
        x.astype(jnp.int32)
        .reshape(_NW, 128, _TT, 8)
        .transpose(2, 0, 3, 1)
    )
    out = _emb_call(x4, token_emb, pos_emb)
    return (
        out.reshape(MAXLEN, 4, _NW, 8, 128)
        .transpose(2, 4, 0, 1, 3)
        .reshape(BATCH, MAXLEN, EMBED)
    )


# speedup vs baseline: 4.0259x; 1.0000x over previous
import functools

import jax
import jax.numpy as jnp
from jax import lax
from jax.experimental import pallas as pl
from jax.experimental.pallas import tpu as pltpu
from jax.experimental.pallas import tpu_sc as plsc

BATCH = 4096
MAXLEN = 200
EMBED = 32

_NC = 2
_NS = 16
_NW = _NC * _NS
_TT = MAXLEN // 8
_TCH = 4
_NCH = MAXLEN // _TCH
_NBLK = _TCH * (EMBED // 8)
_PSTR = 129


def _issue_gather(tok_hbm, idx_all, rows_b, sem, t0):
    pltpu.async_copy(
        tok_hbm.at[idx_all.at[pl.ds(t0 * 128, _TCH * 128)]],
        rows_b,
        sem,
    )


def _wait_gather(tok_hbm, rows_b, sem):
    pltpu.make_async_copy(tok_hbm.at[pl.ds(0, _TCH * 128)], rows_b, sem).wait()


def _transpose_add(rows_b, pos_v, outb, t0):
    lanes = lax.iota(jnp.int32, 16)
    ev = lanes % 8
    zeros = lanes * 0
    for tl in range(_TCH):
        t = t0 + tl
        p0 = pos_v[t, pl.ds(0, 16)]
        p1 = pos_v[t, pl.ds(16, 16)]
        blk0 = tl * 4 + lanes // 8
        blk1 = blk0 + 2

        @plsc.parallel_loop(0, 128, unroll=8)
        def _(b):
            r0 = rows_b[tl * 128 + b, pl.ds(0, 16)]
            r1 = rows_b[tl * 128 + b, pl.ds(16, 16)]
            bv = zeros + b
            plsc.store_scatter(outb, [blk0, ev, bv], r0 + p0)
            plsc.store_scatter(outb, [blk1, ev, bv], r1 + p1)


def _emb_body(x_hbm, tok_hbm, pos_hbm, out_hbm,
              idx_all, pos_v, rows2, out2, gsem0, gsem1, osem0, osem1):
    w = lax.axis_index("s") * _NC + lax.axis_index("c")

    rows_b0 = rows2.at[0]
    rows_b1 = rows2.at[1]
    outb0 = out2.at[0]
    outb1 = out2.at[1]

    pltpu.async_copy(pos_hbm, pos_v, osem0)
    for tt in range(_TT):
        pltpu.async_copy(
            x_hbm.at[tt, w], idx_all.at[pl.ds(tt * 1024, 1024)], gsem0
        )
    for tt in range(_TT):
        pltpu.make_async_copy(
            x_hbm.at[tt, w], idx_all.at[pl.ds(tt * 1024, 1024)], gsem0
        ).wait()
    pltpu.make_async_copy(pos_hbm, pos_v, osem0).wait()

    _issue_gather(tok_hbm, idx_all, rows_b0, gsem0, 0)

    def outer(j, carry):
        ta = 2 * j * _TCH
        tb = ta + _TCH

        _wait_gather(tok_hbm, rows_b0, gsem0)

        @pl.when(j > 0)
        def _():
            pltpu.make_async_copy(
                outb1.at[:, :, pl.ds(0, 128)],
                out_hbm.at[pl.ds(0, _NBLK), w],
                osem1,
            ).wait()

        _issue_gather(tok_hbm, idx_all, rows_b1, gsem1, tb)
        _transpose_add(rows_b0, pos_v, outb0, ta)
        pltpu.async_copy(
            outb0.at[:, :, pl.ds(0, 128)],
            out_hbm.at[pl.ds(ta * 4, _NBLK), w],
            osem0,
        )

        _wait_gather(tok_hbm, rows_b1, gsem1)
        pltpu.make_async_copy(
            outb0.at[:, :, pl.ds(0, 128)],
            out_hbm.at[pl.ds(0, _NBLK), w],
            osem0,
        ).wait()

        @pl.when(j < _NCH // 2 - 1)
        def _():
            _issue_gather(tok_hbm, idx_all, rows_b0, gsem0, tb + _TCH)

        _transpose_add(rows_b1, pos_v, outb1, tb)
        pltpu.async_copy(
            outb1.at[:, :, pl.ds(0, 128)],
            out_hbm.at[pl.ds(tb * 4, _NBLK), w],
            osem1,
        )
        return carry

    lax.fori_loop(0, _NCH // 2, outer, 0)

    pltpu.make_async_copy(
        outb1.at[:, :, pl.ds(0, 128)],
        out_hbm.at[pl.ds(0, _NBLK), w],
        osem1,
    ).wait()


@jax.jit
def _emb_call(x4, token_emb, pos_emb):
    mesh = plsc.VectorSubcoreMesh(core_axis_name="c", subcore_axis_name="s")
    k = functools.partial(
        pl.kernel,
        mesh=mesh,
        out_type=jax.ShapeDtypeStruct((MAXLEN * 4, _NW, 8, 128), jnp.float32),
        scratch_types=[
            pltpu.VMEM((MAXLEN * 128, ), jnp.int32),
            pltpu.VMEM((MAXLEN, EMBED), jnp.float32),
            pltpu.VMEM((2, _TCH * 128, EMBED), jnp.float32),
            pltpu.VMEM((2, _NBLK, 8, _PSTR), jnp.float32),
            pltpu.SemaphoreType.DMA,
            pltpu.SemaphoreType.DMA,
            pltpu.SemaphoreType.DMA,
            pltpu.SemaphoreType.DMA,
        ],
        compiler_params=pltpu.CompilerParams(
            use_tc_tiling_on_sc=False, needs_layout_passes=False
        ),
    )(_emb_body)
    return k(x4, token_emb, pos_emb)


def kernel(x, token_emb, pos_emb):
    x4 = (
        x.astype(jnp.int32)
        .reshape(_NW, 128, _TT, 8)
        .transpose(2, 0, 3, 1)
        .reshape(_TT, _NW, 1024)
    )
    out = _emb_call(x4, token_emb, pos_emb)
    return (
        out.reshape(MAXLEN, 4, _NW, 8, 128)
        .transpose(2, 4, 0, 1, 3)
        .reshape(BATCH, MAXLEN, EMBED)
    )


# speedup vs baseline: 4.1269x; 1.0251x over previous
import functools

import jax
import jax.numpy as jnp
from jax import lax
from jax.experimental import pallas as pl
from jax.experimental.pallas import tpu as pltpu
from jax.experimental.pallas import tpu_sc as plsc

BATCH = 4096
MAXLEN = 200
EMBED = 32
VOCAB_ = 100000

_NC = 2
_NS = 16
_NW = _NC * _NS
_TT = MAXLEN // 8
_TCH = 4
_NCH = MAXLEN // _TCH
_NBLK = _TCH * (EMBED // 8)
_PSTR = 129


def _issue_gather(tok_hbm, idx_all, rows_b, sem, t0):
    for tl in range(_TCH):
        pltpu.async_copy(
            tok_hbm.at[idx_all.at[pl.ds((t0 + tl) * 128, 128)]],
            rows_b.at[pl.ds(tl * 128, 128)],
            sem,
        )


def _wait_gather(tok_hbm, rows_b, sem):
    pltpu.make_async_copy(tok_hbm.at[pl.ds(0, _TCH * 128)], rows_b, sem).wait()


def _transpose_add(rows_b, pos_v, outb, t0):
    lanes = lax.iota(jnp.int32, 16)
    ev = lanes % 8
    zeros = lanes * 0
    for tl in range(_TCH):
        t = t0 + tl
        p0 = pos_v[t, pl.ds(0, 16)]
        p1 = pos_v[t, pl.ds(16, 16)]
        blk0 = tl * 4 + lanes // 8
        blk1 = blk0 + 2

        @plsc.parallel_loop(0, 128, unroll=8)
        def _(b):
            r0 = rows_b[tl * 128 + b, pl.ds(0, 16)]
            r1 = rows_b[tl * 128 + b, pl.ds(16, 16)]
            bv = zeros + b
            plsc.store_scatter(outb, [blk0, ev, bv], r0 + p0)
            plsc.store_scatter(outb, [blk1, ev, bv], r1 + p1)


def _emb_body(x_hbm, tok_hbm, pos_hbm, out_hbm,
              idx_all, pos_v, rows2, out2, gsem0, gsem1, osem0, osem1):
    w = lax.axis_index("s") * _NC + lax.axis_index("c")

    rows_b0 = rows2.at[0]
    rows_b1 = rows2.at[1]
    outb0 = out2.at[0]
    outb1 = out2.at[1]

    pltpu.async_copy(pos_hbm, pos_v, osem0)
    for tt in range(_TT):
        pltpu.async_copy(
            x_hbm.at[tt, w], idx_all.at[pl.ds(tt * 1024, 1024)], gsem0
        )
    for tt in range(_TT):
        pltpu.make_async_copy(
            x_hbm.at[tt, w], idx_all.at[pl.ds(tt * 1024, 1024)], gsem0
        ).wait()
    pltpu.make_async_copy(pos_hbm, pos_v, osem0).wait()

    @plsc.parallel_loop(0, _TT * 1024, step=16, unroll=8)
    def _(i):
        idx_all[pl.ds(i, 16)] = idx_all[pl.ds(i, 16)] * 4

    _issue_gather(tok_hbm, idx_all, rows_b0, gsem0, 0)

    def outer(j, carry):
        ta = 2 * j * _TCH
        tb = ta + _TCH

        _wait_gather(tok_hbm, rows_b0, gsem0)

        @pl.when(j > 0)
        def _():
            pltpu.make_async_copy(
                outb1.at[:, :, pl.ds(0, 128)],
                out_hbm.at[pl.ds(0, _NBLK), w],
                osem1,
            ).wait()

        _issue_gather(tok_hbm, idx_all, rows_b1, gsem1, tb)
        _transpose_add(rows_b0, pos_v, outb0, ta)
        pltpu.async_copy(
            outb0.at[:, :, pl.ds(0, 128)],
            out_hbm.at[pl.ds(ta * 4, _NBLK), w],
            osem0,
        )

        _wait_gather(tok_hbm, rows_b1, gsem1)
        pltpu.make_async_copy(
            outb0.at[:, :, pl.ds(0, 128)],
            out_hbm.at[pl.ds(0, _NBLK), w],
            osem0,
        ).wait()

        @pl.when(j < _NCH // 2 - 1)
        def _():
            _issue_gather(tok_hbm, idx_all, rows_b0, gsem0, tb + _TCH)

        _transpose_add(rows_b1, pos_v, outb1, tb)
        pltpu.async_copy(
            outb1.at[:, :, pl.ds(0, 128)],
            out_hbm.at[pl.ds(tb * 4, _NBLK), w],
            osem1,
        )
        return carry

    lax.fori_loop(0, _NCH // 2, outer, 0)

    pltpu.make_async_copy(
        outb1.at[:, :, pl.ds(0, 128)],
        out_hbm.at[pl.ds(0, _NBLK), w],
        osem1,
    ).wait()


@jax.jit
def _emb_call(x4, tok_pad, pos_emb):
    mesh = plsc.VectorSubcoreMesh(core_axis_name="c", subcore_axis_name="s")
    k = functools.partial(
        pl.kernel,
        mesh=mesh,
        out_type=jax.ShapeDtypeStruct((MAXLEN * 4, _NW, 8, 128), jnp.float32),
        scratch_types=[
            pltpu.VMEM((MAXLEN * 128, ), jnp.int32),
            pltpu.VMEM((MAXLEN, EMBED), jnp.float32),
            pltpu.VMEM((2, _TCH * 128, EMBED), jnp.float32),
            pltpu.VMEM((2, _NBLK, 8, _PSTR), jnp.float32),
            pltpu.SemaphoreType.DMA,
            pltpu.SemaphoreType.DMA,
            pltpu.SemaphoreType.DMA,
            pltpu.SemaphoreType.DMA,
        ],
        compiler_params=pltpu.CompilerParams(
            use_tc_tiling_on_sc=False, needs_layout_passes=False
        ),
    )(_emb_body)
    return k(x4, tok_pad, pos_emb)


def kernel(x, token_emb, pos_emb):
    x4 = (
        x.astype(jnp.int32)
        .reshape(_NW, 128, _TT, 8)
        .transpose(2, 0, 3, 1)
        .reshape(_TT, _NW, 1024)
    )
    tok_pad = jnp.pad(token_emb, ((0, 0), (0, 96))).reshape(4 * VOCAB_, EMBED)
    out = _emb_call(x4, tok_pad, pos_emb)
    return (
        out.reshape(MAXLEN, 4, _NW, 8, 128)
        .transpose(2, 4, 0, 1, 3)
        .reshape(BATCH, MAXLEN, EMBED)
    )
